# baseline (speedup 1.0000x reference)
import functools

import jax
import jax.numpy as jnp
from jax import lax
from jax.experimental import pallas as pl
from jax.experimental.pallas import tpu as pltpu
from jax.experimental.pallas import tpu_sc as plsc

B = 16384
BLK = 4096
D = 32
NCAT = 128
OFF_S, OFF_G, OFF_M = 66, 100, 118
NC, NS = 2, 16
NW = NC * NS
EPW = B // NW


def _sc_gather_body(idx_hbm, tab_hbm, out_hbm, idx_v, rows_v, sem):
    wid = lax.axis_index("s") * NC + lax.axis_index("c")
    base = wid * EPW
    for f in range(4):
        off = f * B + base
        pltpu.sync_copy(idx_hbm.at[pl.ds(off, EPW)], idx_v)
        pltpu.async_copy(tab_hbm.at[idx_v], rows_v, sem).wait()
        pltpu.sync_copy(rows_v, out_hbm.at[pl.ds(off, EPW)])


_sc_gather = pl.kernel(
    _sc_gather_body,
    mesh=plsc.VectorSubcoreMesh(core_axis_name="c", subcore_axis_name="s"),
    out_type=jax.ShapeDtypeStruct((4 * B, NCAT), jnp.float32),
    scratch_types=[
        pltpu.VMEM((EPW,), jnp.int32),
        pltpu.VMEM((EPW, NCAT), jnp.float32),
        pltpu.SemaphoreType.DMA,
    ],
)


def _tc_body(e_ref, cost_ref, time_ref, cw_ref, w1_ref, b1_ref,
             w2_ref, b2_ref, w3_ref, b3_ref, out_ref):
    e = e_ref[0, 0] + e_ref[1, 0] + e_ref[2, 0] + e_ref[3, 0]
    h1 = jnp.dot(e, w1_ref[0:NCAT, :], preferred_element_type=jnp.float32)

    fold = jnp.dot(cw_ref[...], w1_ref[NCAT:192, :],
                   preferred_element_type=jnp.float32)
    ct = jnp.stack([cost_ref[...], time_ref[...]], axis=0)
    h1 = h1 + lax.dot_general(ct, fold[0:2, :], (((0,), (0,)), ((), ())),
                              preferred_element_type=jnp.float32)
    b1pp = b1_ref[...][None, :] + fold[2:3, :] + fold[3:4, :]

    h1 = jnp.maximum(h1 + b1pp, 0.0)
    h2 = jnp.maximum(
        jnp.dot(h1, w2_ref[...], preferred_element_type=jnp.float32)
        + b2_ref[...][None, :],
        0.0)
    out_ref[...] = (jnp.dot(h2, w3_ref[...], preferred_element_type=jnp.float32)
                    + b3_ref[...][None, :])


def kernel(center_idx, subject_idx, grade_idx, method_idx, cost, time,
           center_table, subject_table, grade_table, method_table,
           cost_W, cost_b, time_W, time_b,
           W1, b1, W2, b2, W3, b3):
    idx_all = jnp.concatenate([
        center_idx.astype(jnp.int32),
        subject_idx.astype(jnp.int32) + OFF_S,
        grade_idx.astype(jnp.int32) + OFF_G,
        method_idx.astype(jnp.int32) + OFF_M,
    ])

    tcat = jnp.zeros((NCAT, NCAT), dtype=jnp.float32)
    tcat = tcat.at[0:OFF_S, 0:32].set(center_table)
    tcat = tcat.at[OFF_S:OFF_G, 32:64].set(subject_table)
    tcat = tcat.at[OFF_G:OFF_M, 64:96].set(grade_table)
    tcat = tcat.at[OFF_M:NCAT, 96:128].set(method_table)

    emb = _sc_gather(idx_all, tcat)

    cw = jnp.zeros((4, 64), dtype=jnp.float32)
    cw = cw.at[0, 0:32].set(cost_W[0])
    cw = cw.at[1, 32:64].set(time_W[0])
    cw = cw.at[2, 0:32].set(cost_b)
    cw = cw.at[3, 32:64].set(time_b)

    nb = B // BLK
    e4 = emb.reshape(4, nb, BLK, NCAT)
    vec_spec = pl.BlockSpec((BLK,), lambda i: (i,))
    full = lambda a: pl.BlockSpec(a.shape, lambda i: (0,) * a.ndim)

    return pl.pallas_call(
        _tc_body,
        grid=(nb,),
        in_specs=[pl.BlockSpec((4, 1, BLK, NCAT), lambda i: (0, i, 0, 0)),
                  vec_spec, vec_spec,
                  full(cw), full(W1), full(b1), full(W2), full(b2),
                  full(W3), full(b3)],
        out_specs=pl.BlockSpec((BLK, D), lambda i: (i, 0)),
        out_shape=jax.ShapeDtypeStruct((B, D), jnp.float32),
        compiler_params=pltpu.CompilerParams(
            dimension_semantics=("arbitrary",)),
    )(e4, cost, time, cw, W1, b1, W2, b2, W3, b3)


# speedup vs baseline: 5.7258x; 5.7258x over previous
import functools

import jax
import jax.numpy as jnp
from jax import lax
from jax.experimental import pallas as pl
from jax.experimental.pallas import tpu as pltpu

B = 16384
BLK = 8192
D = 32
V_CENTER, V_SUBJECT, V_GRADE, V_METHOD = 66, 34, 18, 10
OFF_S = V_CENTER
OFF_G = OFF_S + V_SUBJECT
OFF_M = OFF_G + V_GRADE
NCAT = OFF_M + V_METHOD
TEXP_ROWS = 136


def _body(c_ref, s_ref, g_ref, m_ref, cost_ref, time_ref,
          texp_ref, w1_ref, b1_ref, w2_ref, b2_ref, w3_ref, b3_ref,
          out_ref, m_scr):
    @pl.when(pl.program_id(0) == 0)
    def _fold():
        m_scr[...] = jnp.dot(texp_ref[...], w1_ref[...],
                             preferred_element_type=jnp.float32)

    c = c_ref[...]
    s = s_ref[...] + OFF_S
    g = g_ref[...] + OFF_G
    m = m_ref[...] + OFF_M

    col = lax.broadcasted_iota(jnp.int32, (BLK, NCAT), 1)
    onehot = ((col == c[:, None])
              | (col == s[:, None])
              | (col == g[:, None])
              | (col == m[:, None])).astype(jnp.float32)

    ct = jnp.stack([cost_ref[...], time_ref[...]], axis=0)

    mcat = m_scr[0:NCAT, :]
    cwtw = m_scr[NCAT:NCAT + 2, :]
    b1pp = (b1_ref[...][None, :] + m_scr[NCAT + 2:NCAT + 3, :]
            + m_scr[NCAT + 3:NCAT + 4, :])

    h1 = jnp.dot(onehot, mcat, preferred_element_type=jnp.float32)
    h1 = h1 + lax.dot_general(ct, cwtw, (((0,), (0,)), ((), ())),
                              preferred_element_type=jnp.float32)
    h1 = jnp.maximum(h1 + b1pp, 0.0)
    h2 = jnp.maximum(
        jnp.dot(h1, w2_ref[...], preferred_element_type=jnp.float32)
        + b2_ref[...][None, :],
        0.0)
    out_ref[...] = (jnp.dot(h2, w3_ref[...], preferred_element_type=jnp.float32)
                    + b3_ref[...][None, :])


def kernel(center_idx, subject_idx, grade_idx, method_idx, cost, time,
           center_table, subject_table, grade_table, method_table,
           cost_W, cost_b, time_W, time_b,
           W1, b1, W2, b2, W3, b3):
    nb = B // BLK
    ci = center_idx.astype(jnp.int32)
    si = subject_idx.astype(jnp.int32)
    gi = grade_idx.astype(jnp.int32)
    mi = method_idx.astype(jnp.int32)

    texp = jnp.zeros((TEXP_ROWS, 192), dtype=jnp.float32)
    texp = texp.at[0:OFF_S, 0:32].set(center_table)
    texp = texp.at[OFF_S:OFF_G, 32:64].set(subject_table)
    texp = texp.at[OFF_G:OFF_M, 64:96].set(grade_table)
    texp = texp.at[OFF_M:NCAT, 96:128].set(method_table)
    texp = texp.at[NCAT, 128:160].set(cost_W[0])
    texp = texp.at[NCAT + 1, 160:192].set(time_W[0])
    texp = texp.at[NCAT + 2, 128:160].set(cost_b)
    texp = texp.at[NCAT + 3, 160:192].set(time_b)

    vec_spec = pl.BlockSpec((BLK,), lambda i: (i,))
    full = lambda a: pl.BlockSpec(a.shape, lambda i: (0,) * a.ndim)

    return pl.pallas_call(
        _body,
        grid=(nb,),
        in_specs=[vec_spec, vec_spec, vec_spec, vec_spec, vec_spec, vec_spec,
                  full(texp), full(W1), full(b1), full(W2), full(b2),
                  full(W3), full(b3)],
        out_specs=pl.BlockSpec((BLK, D), lambda i: (i, 0)),
        out_shape=jax.ShapeDtypeStruct((B, D), jnp.float32),
        scratch_shapes=[pltpu.VMEM((TEXP_ROWS, 256), jnp.float32)],
        compiler_params=pltpu.CompilerParams(
            dimension_semantics=("arbitrary",)),
    )(ci, si, gi, mi, cost, time, texp, W1, b1, W2, b2, W3, b3)


# speedup vs baseline: 5.8741x; 1.0259x over previous
import functools

import jax
import jax.numpy as jnp
from jax import lax
from jax.experimental import pallas as pl
from jax.experimental.pallas import tpu as pltpu

B = 16384
BLK = 8192
D = 32
V_CENTER, V_SUBJECT, V_GRADE, V_METHOD = 66, 34, 18, 10
OFF_S = V_CENTER
OFF_G = OFF_S + V_SUBJECT
OFF_M = OFF_G + V_GRADE
NCAT = OFF_M + V_METHOD
TEXP_ROWS = 136


def _body(c_ref, s_ref, g_ref, m_ref, cost_ref, time_ref,
          texp_ref, w1_ref, b1_ref, w2_ref, b2_ref, w3_ref, b3_ref,
          out_ref, m_scr):
    @pl.when(pl.program_id(0) == 0)
    def _fold():
        m_scr[...] = jnp.dot(texp_ref[...], w1_ref[...],
                             preferred_element_type=jnp.float32)

    c = c_ref[...]
    s = s_ref[...] + OFF_S
    g = g_ref[...] + OFF_G
    m = m_ref[...] + OFF_M

    col = lax.broadcasted_iota(jnp.int32, (BLK, NCAT), 1)
    onehot = ((col == c[:, None])
              | (col == s[:, None])
              | (col == g[:, None])
              | (col == m[:, None])).astype(jnp.bfloat16)

    ct = jnp.stack([cost_ref[...], time_ref[...]],
                   axis=0).astype(jnp.bfloat16)

    mcat = m_scr[0:NCAT, :].astype(jnp.bfloat16)
    cwtw = m_scr[NCAT:NCAT + 2, :].astype(jnp.bfloat16)
    b1pp = (b1_ref[...][None, :] + m_scr[NCAT + 2:NCAT + 3, :]
            + m_scr[NCAT + 3:NCAT + 4, :])

    h1 = jnp.dot(onehot, mcat, preferred_element_type=jnp.float32)
    h1 = h1 + lax.dot_general(ct, cwtw, (((0,), (0,)), ((), ())),
                              preferred_element_type=jnp.float32)
    h1 = jnp.maximum(h1 + b1pp, 0.0)
    h2 = jnp.maximum(
        jnp.dot(h1.astype(jnp.bfloat16), w2_ref[...].astype(jnp.bfloat16),
                preferred_element_type=jnp.float32)
        + b2_ref[...][None, :],
        0.0)
    out_ref[...] = (jnp.dot(h2.astype(jnp.bfloat16),
                            w3_ref[...].astype(jnp.bfloat16),
                            preferred_element_type=jnp.float32)
                    + b3_ref[...][None, :])


def kernel(center_idx, subject_idx, grade_idx, method_idx, cost, time,
           center_table, subject_table, grade_table, method_table,
           cost_W, cost_b, time_W, time_b,
           W1, b1, W2, b2, W3, b3):
    nb = B // BLK
    ci = center_idx.astype(jnp.int32)
    si = subject_idx.astype(jnp.int32)
    gi = grade_idx.astype(jnp.int32)
    mi = method_idx.astype(jnp.int32)

    texp = jnp.zeros((TEXP_ROWS, 192), dtype=jnp.float32)
    texp = texp.at[0:OFF_S, 0:32].set(center_table)
    texp = texp.at[OFF_S:OFF_G, 32:64].set(subject_table)
    texp = texp.at[OFF_G:OFF_M, 64:96].set(grade_table)
    texp = texp.at[OFF_M:NCAT, 96:128].set(method_table)
    texp = texp.at[NCAT, 128:160].set(cost_W[0])
    texp = texp.at[NCAT + 1, 160:192].set(time_W[0])
    texp = texp.at[NCAT + 2, 128:160].set(cost_b)
    texp = texp.at[NCAT + 3, 160:192].set(time_b)

    vec_spec = pl.BlockSpec((BLK,), lambda i: (i,))
    full = lambda a: pl.BlockSpec(a.shape, lambda i: (0,) * a.ndim)

    return pl.pallas_call(
        _body,
        grid=(nb,),
        in_specs=[vec_spec, vec_spec, vec_spec, vec_spec, vec_spec, vec_spec,
                  full(texp), full(W1), full(b1), full(W2), full(b2),
                  full(W3), full(b3)],
        out_specs=pl.BlockSpec((BLK, D), lambda i: (i, 0)),
        out_shape=jax.ShapeDtypeStruct((B, D), jnp.float32),
        scratch_shapes=[pltpu.VMEM((TEXP_ROWS, 256), jnp.float32)],
        compiler_params=pltpu.CompilerParams(
            dimension_semantics=("arbitrary",)),
    )(ci, si, gi, mi, cost, time, texp, W1, b1, W2, b2, W3, b3)
